# baseline (speedup 1.0000x reference)
import functools

import jax
import jax.numpy as jnp
from jax import lax
from jax.experimental import pallas as pl
from jax.experimental.pallas import tpu as pltpu
from jax.experimental.pallas import tpu_sc as plsc

_NBUF = 3
_CH = 16
_LANES = 16


@functools.cache
def _build_mirror(num_rows: int, row_start: int, table_rows: int, dim: int):
    info = plsc.get_sparse_core_info()
    nw = info.num_cores * info.num_subcores
    half = num_rows // 2
    assert num_rows % (2 * nw * _CH) == 0
    n_chunks = half // (nw * _CH)
    rows_per_w = half // nw
    half_dim = dim // 2
    assert half_dim % _LANES == 0
    assert row_start + num_rows + _CH <= table_rows
    mesh = plsc.VectorSubcoreMesh(core_axis_name="c", subcore_axis_name="s")

    @functools.partial(
        pl.kernel,
        out_type=jax.ShapeDtypeStruct((num_rows, dim), jnp.float32),
        mesh=mesh,
        scratch_types=[
            [pltpu.VMEM((_CH, dim), jnp.float32) for _ in range(_NBUF)],
            [pltpu.VMEM((_CH, dim), jnp.float32) for _ in range(_NBUF)],
            pltpu.VMEM((_CH, dim), jnp.float32),
            [pltpu.SemaphoreType.DMA for _ in range(_NBUF)],
            [pltpu.SemaphoreType.DMA for _ in range(_NBUF)],
            [pltpu.SemaphoreType.DMA for _ in range(_NBUF)],
            pltpu.SemaphoreType.DMA,
        ],
    )
    def copy_kernel(table_hbm, out_hbm, bufs, revs, ext, rsems, bsems, fsems, esem):
        wid = lax.axis_index("s") * info.num_cores + lax.axis_index("c")
        base = half + wid * rows_per_w

        def o0(i):
            return base + i * _CH

        def rd(i):
            src = table_hbm.at[pl.ds(row_start + o0(i), _CH)]
            return pltpu.async_copy(src, bufs[i % _NBUF], rsems[i % _NBUF])

        def wr_back(i):
            dst = out_hbm.at[pl.ds(o0(i), _CH)]
            return pltpu.async_copy(bufs[i % _NBUF], dst, bsems[i % _NBUF])

        def mirror(i, nxt):
            buf, rev = bufs[i % _NBUF], revs[i % _NBUF]
            for j in range(half_dim // _LANES):
                rev[0, pl.ds(j * _LANES, _LANES)] = -nxt[0, pl.ds(j * _LANES, _LANES)]
            for j in range(half_dim // _LANES, dim // _LANES):
                rev[0, pl.ds(j * _LANES, _LANES)] = nxt[0, pl.ds(j * _LANES, _LANES)]

            def body(k, _):
                src = _CH - k
                for j in range(half_dim // _LANES):
                    rev[k, pl.ds(j * _LANES, _LANES)] = -buf[src, pl.ds(j * _LANES, _LANES)]
                for j in range(half_dim // _LANES, dim // _LANES):
                    rev[k, pl.ds(j * _LANES, _LANES)] = buf[src, pl.ds(j * _LANES, _LANES)]
                return 0

            lax.fori_loop(1, _CH, body, 0)

        def wr_front(i):
            f0 = num_rows - o0(i) - _CH
            dst = out_hbm.at[pl.ds(f0, _CH)]
            return pltpu.async_copy(revs[i % _NBUF], dst, fsems[i % _NBUF])

        ext_read = pltpu.async_copy(
            table_hbm.at[pl.ds(row_start + base + rows_per_w, _CH)], ext, esem
        )

        reads = [None] * n_chunks
        backs = [None] * n_chunks
        fronts = [None] * n_chunks
        for i in range(n_chunks):
            if i >= _NBUF:
                backs[i - _NBUF].wait()
            reads[i] = rd(i)
            if i >= 1:
                reads[i - 1].wait()
                backs[i - 1] = wr_back(i - 1)
            if i >= 2:
                if i - 2 >= _NBUF:
                    fronts[i - 2 - _NBUF].wait()
                mirror(i - 2, bufs[(i - 1) % _NBUF])
                fronts[i - 2] = wr_front(i - 2)
        reads[n_chunks - 1].wait()
        backs[n_chunks - 1] = wr_back(n_chunks - 1)
        if n_chunks >= 2:
            if n_chunks - 2 >= _NBUF:
                fronts[n_chunks - 2 - _NBUF].wait()
            mirror(n_chunks - 2, bufs[(n_chunks - 1) % _NBUF])
            fronts[n_chunks - 2] = wr_front(n_chunks - 2)
        ext_read.wait()
        if n_chunks - 1 >= _NBUF:
            fronts[n_chunks - 1 - _NBUF].wait()
        mirror(n_chunks - 1, ext)
        fronts[n_chunks - 1] = wr_front(n_chunks - 1)
        for i in range(max(0, n_chunks - _NBUF), n_chunks):
            backs[i].wait()
            fronts[i].wait()

    return copy_kernel


def kernel(input, weights):
    bsz, seq_len = input.shape
    table_rows, dim = weights.shape
    origin_shift = table_rows // 2
    start = int(-seq_len / 2)
    end = round(seq_len / 2 + 1e-05)
    num_rows = end - start
    row_start = origin_shift + start
    assert row_start + num_rows // 2 == origin_shift
    return _build_mirror(num_rows, row_start, table_rows, dim)(weights)


# speedup vs baseline: 1.3697x; 1.3697x over previous
import functools

import jax
import jax.numpy as jnp
from jax import lax
from jax.experimental import pallas as pl
from jax.experimental.pallas import tpu as pltpu
from jax.experimental.pallas import tpu_sc as plsc

_NBUF = 3
_CHUNK_ROWS = 32


@functools.cache
def _build(num_rows: int, row_start: int, table_rows: int, dim: int):
    info = plsc.get_sparse_core_info()
    nw = info.num_cores * info.num_subcores
    assert num_rows % (nw * _CHUNK_ROWS) == 0
    n_chunks = num_rows // (nw * _CHUNK_ROWS)
    mesh = plsc.VectorSubcoreMesh(core_axis_name="c", subcore_axis_name="s")

    @functools.partial(
        pl.kernel,
        out_type=jax.ShapeDtypeStruct((num_rows, dim), jnp.float32),
        mesh=mesh,
        scratch_types=[
            [pltpu.VMEM((_CHUNK_ROWS, dim), jnp.float32) for _ in range(_NBUF)],
            [pltpu.SemaphoreType.DMA for _ in range(_NBUF)],
            [pltpu.SemaphoreType.DMA for _ in range(_NBUF)],
        ],
    )
    def copy_kernel(table_hbm, out_hbm, bufs, rsems, wsems):
        wid = lax.axis_index("s") * info.num_cores + lax.axis_index("c")

        def chunk_row(i):
            return (wid + i * nw) * _CHUNK_ROWS

        def rd(i, b):
            src = table_hbm.at[pl.ds(row_start + chunk_row(i), _CHUNK_ROWS)]
            return pltpu.async_copy(src, bufs[b], rsems[b])

        def wr(i, b):
            dst = out_hbm.at[pl.ds(chunk_row(i), _CHUNK_ROWS)]
            return pltpu.async_copy(bufs[b], dst, wsems[b])

        reads = [None] * n_chunks
        writes = [None] * n_chunks
        for i in range(n_chunks):
            b = i % _NBUF
            if i >= _NBUF:
                writes[i - _NBUF].wait()
            reads[i] = rd(i, b)
            if i >= 1:
                reads[i - 1].wait()
                writes[i - 1] = wr(i - 1, (i - 1) % _NBUF)
        reads[n_chunks - 1].wait()
        writes[n_chunks - 1] = wr(n_chunks - 1, (n_chunks - 1) % _NBUF)
        for i in range(max(0, n_chunks - _NBUF), n_chunks):
            writes[i].wait()

    return copy_kernel


def kernel(input, weights):
    bsz, seq_len = input.shape
    table_rows, dim = weights.shape
    origin_shift = table_rows // 2
    start = int(-seq_len / 2)
    end = round(seq_len / 2 + 1e-05)
    num_rows = end - start
    row_start = origin_shift + start
    return _build(num_rows, row_start, table_rows, dim)(weights)


# speedup vs baseline: 1.4049x; 1.0257x over previous
import functools

import jax
import jax.numpy as jnp
from jax import lax
from jax.experimental import pallas as pl
from jax.experimental.pallas import tpu as pltpu
from jax.experimental.pallas import tpu_sc as plsc

_NBUF = 5
_CHUNK_ROWS = 16


@functools.cache
def _build(num_rows: int, row_start: int, table_rows: int, dim: int):
    info = plsc.get_sparse_core_info()
    nw = info.num_cores * info.num_subcores
    assert num_rows % (nw * _CHUNK_ROWS) == 0
    n_chunks = num_rows // (nw * _CHUNK_ROWS)
    mesh = plsc.VectorSubcoreMesh(core_axis_name="c", subcore_axis_name="s")

    @functools.partial(
        pl.kernel,
        out_type=jax.ShapeDtypeStruct((num_rows, dim), jnp.float32),
        mesh=mesh,
        scratch_types=[
            [pltpu.VMEM((_CHUNK_ROWS, dim), jnp.float32) for _ in range(_NBUF)],
            [pltpu.SemaphoreType.DMA for _ in range(_NBUF)],
            [pltpu.SemaphoreType.DMA for _ in range(_NBUF)],
        ],
    )
    def copy_kernel(table_hbm, out_hbm, bufs, rsems, wsems):
        wid = lax.axis_index("s") * info.num_cores + lax.axis_index("c")

        def chunk_row(i):
            return (wid + i * nw) * _CHUNK_ROWS

        def rd(i, b):
            src = table_hbm.at[pl.ds(row_start + chunk_row(i), _CHUNK_ROWS)]
            return pltpu.async_copy(src, bufs[b], rsems[b])

        def wr(i, b):
            dst = out_hbm.at[pl.ds(chunk_row(i), _CHUNK_ROWS)]
            return pltpu.async_copy(bufs[b], dst, wsems[b])

        reads = [None] * n_chunks
        writes = [None] * n_chunks
        for i in range(n_chunks):
            b = i % _NBUF
            if i >= _NBUF:
                writes[i - _NBUF].wait()
            reads[i] = rd(i, b)
            if i >= 2:
                reads[i - 2].wait()
                writes[i - 2] = wr(i - 2, (i - 2) % _NBUF)
        for i in range(max(0, n_chunks - 2), n_chunks):
            reads[i].wait()
            writes[i] = wr(i, i % _NBUF)
        for i in range(max(0, n_chunks - _NBUF), n_chunks):
            writes[i].wait()

    return copy_kernel


def kernel(input, weights):
    bsz, seq_len = input.shape
    table_rows, dim = weights.shape
    origin_shift = table_rows // 2
    start = int(-seq_len / 2)
    end = round(seq_len / 2 + 1e-05)
    num_rows = end - start
    row_start = origin_shift + start
    return _build(num_rows, row_start, table_rows, dim)(weights)


# speedup vs baseline: 1.4124x; 1.0054x over previous
import functools

import jax
import jax.numpy as jnp
from jax import lax
from jax.experimental import pallas as pl
from jax.experimental.pallas import tpu as pltpu
from jax.experimental.pallas import tpu_sc as plsc

_NBUF = 6
_CHUNK_ROWS = 16


@functools.cache
def _build(num_rows: int, row_start: int, table_rows: int, dim: int):
    info = plsc.get_sparse_core_info()
    nw = info.num_cores * info.num_subcores
    assert num_rows % (nw * _CHUNK_ROWS) == 0
    n_chunks = num_rows // (nw * _CHUNK_ROWS)
    mesh = plsc.VectorSubcoreMesh(core_axis_name="c", subcore_axis_name="s")

    @functools.partial(
        pl.kernel,
        out_type=jax.ShapeDtypeStruct((num_rows, dim), jnp.float32),
        mesh=mesh,
        scratch_types=[
            [pltpu.VMEM((_CHUNK_ROWS, dim), jnp.float32) for _ in range(_NBUF)],
            [pltpu.SemaphoreType.DMA for _ in range(_NBUF)],
            [pltpu.SemaphoreType.DMA for _ in range(_NBUF)],
        ],
    )
    def copy_kernel(table_hbm, out_hbm, bufs, rsems, wsems):
        wid = lax.axis_index("s") * info.num_cores + lax.axis_index("c")

        def chunk_row(i):
            return (wid + i * nw) * _CHUNK_ROWS

        def rd(i, b):
            src = table_hbm.at[pl.ds(row_start + chunk_row(i), _CHUNK_ROWS)]
            return pltpu.async_copy(src, bufs[b], rsems[b])

        def wr(i, b):
            dst = out_hbm.at[pl.ds(chunk_row(i), _CHUNK_ROWS)]
            return pltpu.async_copy(bufs[b], dst, wsems[b])

        reads = [None] * n_chunks
        writes = [None] * n_chunks
        for i in range(n_chunks):
            b = i % _NBUF
            if i >= _NBUF:
                writes[i - _NBUF].wait()
            reads[i] = rd(i, b)
            if i >= 3:
                reads[i - 3].wait()
                writes[i - 3] = wr(i - 3, (i - 3) % _NBUF)
        for i in range(max(0, n_chunks - 3), n_chunks):
            reads[i].wait()
            writes[i] = wr(i, i % _NBUF)
        for i in range(max(0, n_chunks - _NBUF), n_chunks):
            writes[i].wait()

    return copy_kernel


def kernel(input, weights):
    bsz, seq_len = input.shape
    table_rows, dim = weights.shape
    origin_shift = table_rows // 2
    start = int(-seq_len / 2)
    end = round(seq_len / 2 + 1e-05)
    num_rows = end - start
    row_start = origin_shift + start
    return _build(num_rows, row_start, table_rows, dim)(weights)


# speedup vs baseline: 1.4201x; 1.0054x over previous
import functools

import jax
import jax.numpy as jnp
from jax import lax
from jax.experimental import pallas as pl
from jax.experimental.pallas import tpu as pltpu
from jax.experimental.pallas import tpu_sc as plsc

_NBUF = 7
_CHUNK_ROWS = 16


@functools.cache
def _build(num_rows: int, row_start: int, table_rows: int, dim: int):
    info = plsc.get_sparse_core_info()
    nw = info.num_cores * info.num_subcores
    assert num_rows % (nw * _CHUNK_ROWS) == 0
    n_chunks = num_rows // (nw * _CHUNK_ROWS)
    mesh = plsc.VectorSubcoreMesh(core_axis_name="c", subcore_axis_name="s")

    @functools.partial(
        pl.kernel,
        out_type=jax.ShapeDtypeStruct((num_rows, dim), jnp.float32),
        mesh=mesh,
        scratch_types=[
            [pltpu.VMEM((_CHUNK_ROWS, dim), jnp.float32) for _ in range(_NBUF)],
            [pltpu.SemaphoreType.DMA for _ in range(_NBUF)],
            [pltpu.SemaphoreType.DMA for _ in range(_NBUF)],
        ],
    )
    def copy_kernel(table_hbm, out_hbm, bufs, rsems, wsems):
        wid = lax.axis_index("s") * info.num_cores + lax.axis_index("c")

        def chunk_row(i):
            return (wid + i * nw) * _CHUNK_ROWS

        def rd(i, b):
            src = table_hbm.at[pl.ds(row_start + chunk_row(i), _CHUNK_ROWS)]
            return pltpu.async_copy(src, bufs[b], rsems[b])

        def wr(i, b):
            dst = out_hbm.at[pl.ds(chunk_row(i), _CHUNK_ROWS)]
            return pltpu.async_copy(bufs[b], dst, wsems[b])

        reads = [None] * n_chunks
        writes = [None] * n_chunks
        for i in range(n_chunks):
            b = i % _NBUF
            if i >= _NBUF:
                writes[i - _NBUF].wait()
            reads[i] = rd(i, b)
            if i >= 4:
                reads[i - 4].wait()
                writes[i - 4] = wr(i - 4, (i - 4) % _NBUF)
        for i in range(max(0, n_chunks - 4), n_chunks):
            reads[i].wait()
            writes[i] = wr(i, i % _NBUF)
        for i in range(max(0, n_chunks - _NBUF), n_chunks):
            writes[i].wait()

    return copy_kernel


def kernel(input, weights):
    bsz, seq_len = input.shape
    table_rows, dim = weights.shape
    origin_shift = table_rows // 2
    start = int(-seq_len / 2)
    end = round(seq_len / 2 + 1e-05)
    num_rows = end - start
    row_start = origin_shift + start
    return _build(num_rows, row_start, table_rows, dim)(weights)
